# Initial kernel scaffold; baseline (speedup 1.0000x reference)
import functools

import jax
import jax.numpy as jnp
from jax import lax
from jax.experimental import pallas as pl
from jax.experimental.pallas import tpu as pltpu
from jax.experimental.pallas import tpu_sc as plsc

EMB_IN = 50
EMB_OUT = 128
NW = 32
CHUNK = 128


def _gather_body(table_hbm, idx_hbm, out_hbm, idx_v, rows_v, sem, *, nchunk):
    wid = lax.axis_index("s") * 2 + lax.axis_index("c")
    per_w = nchunk * CHUNK
    pltpu.sync_copy(idx_hbm.at[wid], idx_v)

    def body(j, _):
        pltpu.async_copy(table_hbm.at[idx_v.at[j]], rows_v, sem).wait()
        base = wid * per_w + j * CHUNK
        pltpu.sync_copy(rows_v, out_hbm.at[pl.ds(base, CHUNK)])
        return 0

    lax.fori_loop(0, nchunk, body, 0)


def _sc_gather(table, idx_flat):
    n = idx_flat.shape[0]
    per_w = n // NW
    nchunk = per_w // CHUNK
    idx3 = idx_flat.reshape(NW, nchunk, CHUNK)
    mesh = plsc.VectorSubcoreMesh(core_axis_name="c", subcore_axis_name="s")
    gather = pl.kernel(
        functools.partial(_gather_body, nchunk=nchunk),
        out_type=jax.ShapeDtypeStruct((n, EMB_IN), jnp.float32),
        scratch_types=[
            pltpu.VMEM((nchunk, CHUNK), jnp.int32),
            pltpu.VMEM((CHUNK, EMB_IN), jnp.float32),
            pltpu.SemaphoreType.DMA,
        ],
        mesh=mesh,
    )
    return gather(table, idx3)


def _mm_body(emb_ref, w_ref, b_ref, out_ref):
    out_ref[...] = (
        jnp.dot(emb_ref[...], w_ref[...], preferred_element_type=jnp.float32)
        + b_ref[...]
    )


def _tc_project(emb, W, b):
    n = emb.shape[0]
    bm = 1024
    mm = pl.pallas_call(
        _mm_body,
        grid=(n // bm,),
        in_specs=[
            pl.BlockSpec((bm, EMB_IN), lambda i: (i, 0)),
            pl.BlockSpec((EMB_IN, EMB_OUT), lambda i: (0, 0)),
            pl.BlockSpec((1, EMB_OUT), lambda i: (0, 0)),
        ],
        out_specs=pl.BlockSpec((bm, EMB_OUT), lambda i: (i, 0)),
        out_shape=jax.ShapeDtypeStruct((n, EMB_OUT), jnp.float32),
    )
    return mm(emb, W, b.reshape(1, EMB_OUT))


def kernel(observations, table, W, b):
    batch, seq = observations.shape
    idx_flat = observations.reshape(-1).astype(jnp.int32)
    emb = _sc_gather(table, idx_flat)
    out = _tc_project(emb, W, b)
    return out.reshape(batch, seq, EMB_OUT)


# speedup vs baseline: 8.5147x; 8.5147x over previous
import functools

import jax
import jax.numpy as jnp
from jax import lax
from jax.experimental import pallas as pl
from jax.experimental.pallas import tpu as pltpu
from jax.experimental.pallas import tpu_sc as plsc

EMB_IN = 50
EMB_OUT = 128
NW = 32
CHUNK = 128


def _proj_body(t_ref, w_ref, b_ref, p_ref):
    p_ref[...] = (
        jnp.dot(t_ref[...], w_ref[...], preferred_element_type=jnp.float32)
        + b_ref[...]
    )


def _tc_project_table(table, W, b):
    v = table.shape[0]
    bm = 4000
    mm = pl.pallas_call(
        _proj_body,
        grid=(v // bm,),
        in_specs=[
            pl.BlockSpec((bm, EMB_IN), lambda i: (i, 0)),
            pl.BlockSpec((EMB_IN, EMB_OUT), lambda i: (0, 0)),
            pl.BlockSpec((1, EMB_OUT), lambda i: (0, 0)),
        ],
        out_specs=pl.BlockSpec((bm, EMB_OUT), lambda i: (i, 0)),
        out_shape=jax.ShapeDtypeStruct((v, EMB_OUT), jnp.float32),
    )
    return mm(table, W, b.reshape(1, EMB_OUT))


NSLOT = 6


def _gather_body(p_hbm, idx_hbm, out_hbm, idx_v, buf_v, gsem, osem, *, nchunk):
    wid = lax.axis_index("s") * 2 + lax.axis_index("c")
    per_w = nchunk * CHUNK
    pltpu.sync_copy(idx_hbm.at[pl.ds(wid * nchunk, nchunk)], idx_v)

    def gather_start(j, slot):
        pltpu.async_copy(p_hbm.at[idx_v.at[j]], buf_v.at[slot], gsem.at[slot])

    def gather_wait(slot):
        pltpu.make_async_copy(p_hbm.at[idx_v.at[0]], buf_v.at[slot],
                              gsem.at[slot]).wait()

    def out_start(j, slot):
        base = wid * per_w + j * CHUNK
        pltpu.async_copy(buf_v.at[slot], out_hbm.at[pl.ds(base, CHUNK)],
                         osem.at[slot])

    def out_wait(slot):
        base = wid * per_w
        pltpu.make_async_copy(buf_v.at[slot], out_hbm.at[pl.ds(base, CHUNK)],
                              osem.at[slot]).wait()

    for j in range(NSLOT - 1):
        gather_start(j, j)

    def body(j, _):
        slot = lax.rem(j, NSLOT)
        gather_wait(slot)
        out_start(j, slot)

        @pl.when(j + NSLOT - 1 < nchunk)
        def _():
            s2 = lax.rem(j + NSLOT - 1, NSLOT)

            @pl.when(j > 0)
            def _():
                out_wait(s2)
            gather_start(j + NSLOT - 1, s2)

        return 0

    lax.fori_loop(0, nchunk, body, 0)

    for k in range(nchunk - NSLOT, nchunk):
        out_wait(k % NSLOT)


def _sc_gather(p, idx2):
    nrow = idx2.shape[0] * CHUNK
    nchunk = idx2.shape[0] // NW
    mesh = plsc.VectorSubcoreMesh(core_axis_name="c", subcore_axis_name="s")
    gather = pl.kernel(
        functools.partial(_gather_body, nchunk=nchunk),
        out_type=jax.ShapeDtypeStruct((nrow, EMB_OUT), jnp.float32),
        scratch_types=[
            pltpu.VMEM((nchunk, CHUNK), jnp.int32),
            pltpu.VMEM((NSLOT, CHUNK, EMB_OUT), jnp.float32),
            pltpu.SemaphoreType.DMA((NSLOT,)),
            pltpu.SemaphoreType.DMA((NSLOT,)),
        ],
        mesh=mesh,
        compiler_params=pltpu.CompilerParams(use_tc_tiling_on_sc=False),
    )
    return gather(p, idx2)


def kernel(observations, table, W, b):
    batch, seq = observations.shape
    n = batch * seq
    p = _tc_project_table(table, W, b)
    idx2 = observations.reshape(n // CHUNK, CHUNK).astype(jnp.int32)
    out = _sc_gather(p, idx2)
    return out.reshape(batch, seq, EMB_OUT)
